# baseline (speedup 1.0000x reference)
import functools

import jax
import jax.numpy as jnp
from jax import lax
from jax.experimental import pallas as pl
from jax.experimental.pallas import tpu as pltpu
from jax.experimental.pallas import tpu_sc as plsc

E_LAYERS = 6
NUM_EXPERTS = 8
NUM_HEADS = 12
HEAD_DIM = 64
B = 128
D = NUM_HEADS * NUM_EXPERTS * HEAD_DIM
NC = 2
NS = 16
NW = NC * NS
ROWS_PER_W = (2 * B) // NW


_mesh = plsc.VectorSubcoreMesh(core_axis_name="c", subcore_axis_name="s")


@functools.partial(
    pl.kernel,
    mesh=_mesh,
    out_type=[
        jax.ShapeDtypeStruct((B, D), jnp.float32),
        jax.ShapeDtypeStruct((B, D), jnp.float32),
    ],
    scratch_types=[
        pltpu.VMEM((1,), jnp.int32),
        pltpu.VMEM((1, D), jnp.float32),
        pltpu.SemaphoreType.DMA,
    ],
    compiler_params=pltpu.CompilerParams(skip_device_barrier=True),
)
def _gather_broadcast(pk_hbm, pv_hbm, idx_hbm, ek_hbm, ev_hbm,
                      idx_v, row_v, sem):
    wid = lax.axis_index("s") * NC + lax.axis_index("c")
    pltpu.sync_copy(idx_hbm, idx_v)

    def _bcast_rows(table_hbm, out_hbm, base):
        pltpu.async_copy(table_hbm.at[idx_v], row_v, sem).wait()
        copies = [
            pltpu.async_copy(row_v, out_hbm.at[pl.ds(base + r, 1)], sem)
            for r in range(ROWS_PER_W)
        ]
        for c in copies:
            c.wait()

    @pl.when(wid < NS)
    def _ek():
        _bcast_rows(pk_hbm, ek_hbm, wid * ROWS_PER_W)

    @pl.when(wid >= NS)
    def _ev():
        _bcast_rows(pv_hbm, ev_hbm, (wid - NS) * ROWS_PER_W)


def kernel(x_querry, l, x_block, e_pk, e_pv):
    one = x_querry[0, 0] * jnp.float32(0.0) + jnp.float32(1.0)
    split = 24
    xb_a = x_block[:split] * one
    pk2 = e_pk.reshape(E_LAYERS, D)
    pv2 = e_pv.reshape(E_LAYERS, D)
    idx = jnp.asarray(l, dtype=jnp.int32).reshape(1)
    idx, pk2, pv2 = lax.optimization_barrier((idx, pk2, pv2, xb_a))[:3]
    ek2, ev2 = _gather_broadcast(pk2, pv2, idx)
    xb_b = x_block[split:] * one
    xb = jnp.concatenate([xb_a, xb_b], axis=0)
    Ek = ek2.reshape(B, NUM_HEADS, NUM_EXPERTS, HEAD_DIM)
    Ev = ev2.reshape(B, NUM_HEADS, NUM_EXPERTS, HEAD_DIM)
    eps_decay = jnp.full((NUM_HEADS, NUM_EXPERTS), 2.0, dtype=jnp.float32)
    loss = jnp.float32(0.0)
    return (Ek, Ev, eps_decay, loss, xb)


# speedup vs baseline: 1.4361x; 1.4361x over previous
import functools

import jax
import jax.numpy as jnp
from jax import lax
from jax.experimental import pallas as pl
from jax.experimental.pallas import tpu as pltpu
from jax.experimental.pallas import tpu_sc as plsc

E_LAYERS = 6
NUM_EXPERTS = 8
NUM_HEADS = 12
HEAD_DIM = 64
B = 128
D = NUM_HEADS * NUM_EXPERTS * HEAD_DIM
NC = 2
NS = 16


_mesh = plsc.VectorSubcoreMesh(core_axis_name="c", subcore_axis_name="s")


@functools.partial(
    pl.kernel,
    mesh=_mesh,
    out_type=jax.ShapeDtypeStruct((2, D), jnp.float32),
    scratch_types=[
        pltpu.VMEM((1,), jnp.int32),
        pltpu.VMEM((1, D), jnp.float32),
        pltpu.SemaphoreType.DMA,
    ],
)
def _gather_rows(pk_hbm, pv_hbm, idx_hbm, kv_hbm, idx_v, row_v, sem):
    wid = lax.axis_index("s") * NC + lax.axis_index("c")

    @pl.when(wid < 2)
    def _gather():
        pltpu.sync_copy(idx_hbm, idx_v)

        @pl.when(wid == 0)
        def _pk():
            pltpu.async_copy(pk_hbm.at[idx_v], row_v, sem).wait()
            pltpu.sync_copy(row_v, kv_hbm.at[pl.ds(0, 1)])

        @pl.when(wid == 1)
        def _pv():
            pltpu.async_copy(pv_hbm.at[idx_v], row_v, sem).wait()
            pltpu.sync_copy(row_v, kv_hbm.at[pl.ds(1, 1)])


def _bcast_body(kv_ref, ek_ref, ev_ref, eps_ref):
    k = kv_ref[0, :].reshape(1, NUM_HEADS, NUM_EXPERTS, HEAD_DIM)
    v = kv_ref[1, :].reshape(1, NUM_HEADS, NUM_EXPERTS, HEAD_DIM)
    ek_ref[...] = jnp.broadcast_to(k, (B, NUM_HEADS, NUM_EXPERTS, HEAD_DIM))
    ev_ref[...] = jnp.broadcast_to(v, (B, NUM_HEADS, NUM_EXPERTS, HEAD_DIM))
    eps_ref[...] = jnp.full((NUM_HEADS, NUM_EXPERTS), 2.0, jnp.float32)


def _tc_broadcast(kv):
    return pl.pallas_call(
        _bcast_body,
        out_shape=[
            jax.ShapeDtypeStruct((B, NUM_HEADS, NUM_EXPERTS, HEAD_DIM), jnp.float32),
            jax.ShapeDtypeStruct((B, NUM_HEADS, NUM_EXPERTS, HEAD_DIM), jnp.float32),
            jax.ShapeDtypeStruct((NUM_HEADS, NUM_EXPERTS), jnp.float32),
        ],
    )(kv)


def kernel(x_querry, l, x_block, e_pk, e_pv):
    pk2 = e_pk.reshape(E_LAYERS, D)
    pv2 = e_pv.reshape(E_LAYERS, D)
    idx = jnp.asarray(l, dtype=jnp.int32).reshape(1)
    kv = _gather_rows(pk2, pv2, idx)
    one = x_querry[0, 0] * jnp.float32(0.0) + jnp.float32(1.0)
    xb = x_block * one
    Ek, Ev, eps_decay = _tc_broadcast(kv)
    loss = jnp.float32(0.0)
    return (Ek, Ev, eps_decay, loss, xb)


# speedup vs baseline: 1.6442x; 1.1449x over previous
import functools

import jax
import jax.numpy as jnp
from jax import lax
from jax.experimental import pallas as pl
from jax.experimental.pallas import tpu as pltpu
from jax.experimental.pallas import tpu_sc as plsc

E_LAYERS = 6
NUM_EXPERTS = 8
NUM_HEADS = 12
HEAD_DIM = 64
B = 128
D = NUM_HEADS * NUM_EXPERTS * HEAD_DIM
NC = 2
NS = 16
NW = NC * NS
ROWS_PER_W = (2 * B) // NW


_mesh = plsc.VectorSubcoreMesh(core_axis_name="c", subcore_axis_name="s")


@functools.partial(
    pl.kernel,
    mesh=_mesh,
    out_type=[
        jax.ShapeDtypeStruct((B, D), jnp.float32),
        jax.ShapeDtypeStruct((B, D), jnp.float32),
    ],
    scratch_types=[
        pltpu.VMEM((1,), jnp.int32),
        pltpu.VMEM((1, D), jnp.float32),
        pltpu.SemaphoreType.DMA,
    ],
)
def _gather_broadcast(pk_hbm, pv_hbm, idx_hbm, ek_hbm, ev_hbm,
                      idx_v, row_v, sem):
    wid = lax.axis_index("s") * NC + lax.axis_index("c")
    pltpu.sync_copy(idx_hbm, idx_v)

    def _bcast_rows(table_hbm, out_hbm, base):
        pltpu.async_copy(table_hbm.at[idx_v], row_v, sem).wait()
        copies = [
            pltpu.async_copy(row_v, out_hbm.at[pl.ds(base + r, 1)], sem)
            for r in range(ROWS_PER_W)
        ]
        for c in copies:
            c.wait()

    @pl.when(wid < NS)
    def _ek():
        _bcast_rows(pk_hbm, ek_hbm, wid * ROWS_PER_W)

    @pl.when(wid >= NS)
    def _ev():
        _bcast_rows(pv_hbm, ev_hbm, (wid - NS) * ROWS_PER_W)


def kernel(x_querry, l, x_block, e_pk, e_pv):
    pk2 = e_pk.reshape(E_LAYERS, D)
    pv2 = e_pv.reshape(E_LAYERS, D)
    idx = jnp.asarray(l, dtype=jnp.int32).reshape(1)
    ek2, ev2 = _gather_broadcast(pk2, pv2, idx)
    one = x_querry[0, 0] * jnp.float32(0.0) + jnp.float32(1.0)
    xb = x_block * one
    Ek = ek2.reshape(B, NUM_HEADS, NUM_EXPERTS, HEAD_DIM)
    Ev = ev2.reshape(B, NUM_HEADS, NUM_EXPERTS, HEAD_DIM)
    eps_decay = jnp.full((NUM_HEADS, NUM_EXPERTS), 2.0, dtype=jnp.float32)
    loss = jnp.float32(0.0)
    return (Ek, Ev, eps_decay, loss, xb)


# speedup vs baseline: 1.6700x; 1.0157x over previous
import functools

import jax
import jax.numpy as jnp
from jax import lax
from jax.experimental import pallas as pl
from jax.experimental.pallas import tpu as pltpu
from jax.experimental.pallas import tpu_sc as plsc

E_LAYERS = 6
NUM_EXPERTS = 8
NUM_HEADS = 12
HEAD_DIM = 64
B = 128
D = NUM_HEADS * NUM_EXPERTS * HEAD_DIM


_smesh = plsc.ScalarSubcoreMesh(axis_name="c", num_cores=2)


@functools.partial(
    pl.kernel,
    mesh=_smesh,
    out_type=[
        jax.ShapeDtypeStruct((B, D), jnp.float32),
        jax.ShapeDtypeStruct((B, D), jnp.float32),
    ],
    scratch_types=[
        pltpu.VMEM_SHARED((1, D), jnp.float32),
        pltpu.SemaphoreType.DMA,
    ],
)
def _scs_broadcast(kv_hbm, ek_hbm, ev_hbm, row_sp, sem):
    cid = lax.axis_index("c")

    def _fan_out(src_row, out_hbm):
        pltpu.sync_copy(kv_hbm.at[pl.ds(src_row, 1)], row_sp)
        copies = [
            pltpu.async_copy(row_sp, out_hbm.at[pl.ds(r, 1)], sem)
            for r in range(B)
        ]
        for c in copies:
            c.wait()

    @pl.when(cid == 0)
    def _ek():
        _fan_out(0, ek_hbm)

    @pl.when(cid == 1)
    def _ev():
        _fan_out(1, ev_hbm)


def kernel(x_querry, l, x_block, e_pk, e_pv):
    kv = jnp.stack([
        lax.dynamic_index_in_dim(e_pk, l, 0, keepdims=False).reshape(D),
        lax.dynamic_index_in_dim(e_pv, l, 0, keepdims=False).reshape(D),
    ])
    ek2, ev2 = _scs_broadcast(kv)
    one = x_querry[0, 0] * jnp.float32(0.0) + jnp.float32(1.0)
    xb = x_block * one
    Ek = ek2.reshape(B, NUM_HEADS, NUM_EXPERTS, HEAD_DIM)
    Ev = ev2.reshape(B, NUM_HEADS, NUM_EXPERTS, HEAD_DIM)
    eps_decay = jnp.full((NUM_HEADS, NUM_EXPERTS), 2.0, dtype=jnp.float32)
    loss = jnp.float32(0.0)
    return (Ek, Ev, eps_decay, loss, xb)
